# baseline (speedup 1.0000x reference)
import functools

import jax
import jax.numpy as jnp
from jax import lax
from jax.experimental import pallas as pl
from jax.experimental.pallas import tpu as pltpu
from jax.experimental.pallas import tpu_sc as plsc

N_NODES = 10000
N_PAD = 10240
SEG = 640
SEG_OFF = 8000
N_EDGES = 320000
CH = 128
K = 80
R3 = 3
R4 = 4
NC = 2
NS = 16
EPS = 1e-5


def _fast_rsqrt(d):
    i = lax.bitcast_convert_type(d, jnp.int32)
    y = lax.bitcast_convert_type(jnp.int32(0x5F3759DF) - (i >> 1), jnp.float32)
    half = jnp.float32(0.5) * d
    for _ in range(3):
        y = y * (jnp.float32(1.5) - half * y * y)
    return y


def _sc_body(pk_hbm, ew_hbm, x_hbm,
             pout_hbm, dis_hbm,
             pk, pkslab, ewv, normv, rows, dis_t,
             deg_sh, agg_sh,
             gsem, ssem, isem):
    cid = lax.axis_index("c")
    sid = lax.axis_index("s")
    zero16 = jnp.zeros((16,), jnp.float32)
    last = NS - 1
    nrow_last = N_NODES - last * SEG
    max_ch = N_EDGES // K - 1

    @pl.loop(0, K * CH // 16)
    def _zrows(i):
        rows[0, i // (CH // 16), pl.ds((i % (CH // 16)) * 16, 16)] = zero16

    @pl.loop(0, SEG // 16)
    def _zseg(i):
        dis_t[pl.ds(SEG_OFF + i * 16, 16)] = zero16

    pltpu.async_copy(dis_t.at[pl.ds(SEG_OFF, SEG)],
                     deg_sh.at[pl.ds(sid * SEG, SEG)], ssem)
    nz = jnp.where(sid < last, SEG // K, nrow_last // K)
    for i in range(SEG // K):
        @pl.when(i < nz)
        def _():
            pltpu.async_copy(rows.at[0],
                             agg_sh.at[pl.ds(sid * SEG + i * K, K)], ssem)
    pltpu.make_async_copy(dis_t.at[pl.ds(SEG_OFF, SEG)],
                          deg_sh.at[pl.ds(0, SEG)], ssem).wait()
    for i in range(SEG // K):
        @pl.when(i < nz)
        def _():
            pltpu.make_async_copy(rows.at[0],
                                  agg_sh.at[pl.ds(0, K)], ssem).wait()

    plsc.subcore_barrier()

    SLB = 10
    deg_slabs = N_EDGES // NS // K // SLB
    deg_sb = sid * deg_slabs
    max_slab = N_EDGES // K // SLB - 1

    def _dld(s, bu):
        ss = jnp.minimum(s, max_slab)
        pltpu.async_copy(pk_hbm.at[1, pl.ds(ss * SLB, SLB)],
                         pkslab.at[bu], isem)
        pltpu.async_copy(ew_hbm.at[pl.ds(ss * SLB * K, SLB * K)],
                         dis_t.at[pl.ds(bu * SLB * K, SLB * K)], isem)

    def _dwait_ld(bu):
        pltpu.make_async_copy(pk_hbm.at[1, pl.ds(0, SLB)],
                              pkslab.at[bu], isem).wait()
        pltpu.make_async_copy(ew_hbm.at[pl.ds(0, SLB * K)],
                              dis_t.at[pl.ds(0, SLB * K)], isem).wait()

    def _dscat(bu):
        for j in range(SLB):
            pltpu.async_copy(dis_t.at[pl.ds((bu * SLB + j) * K, K)],
                             deg_sh.at[pkslab.at[bu, j, 0]], ssem, add=True)

    def _ddrain():
        for j in range(SLB):
            pltpu.make_async_copy(dis_t.at[pl.ds(0, K)],
                                  deg_sh.at[pkslab.at[0, 0, 0]], ssem).wait()

    def _dslab(s, bu, drain):
        _dwait_ld(bu)
        _dscat(bu)
        if drain:
            _ddrain()
        _dld(deg_sb + s + 2, (bu + 2) % 3)

    _dld(deg_sb + 0, 0)
    _dld(deg_sb + 1, 1)
    _dslab(0, 0, False)
    for s in (1, 2, 3):
        _dslab(s, s % 3, True)

    @pl.loop(0, (deg_slabs - 4) // 3)
    def _deg(g):
        for i in range(3):
            _dslab(4 + g * 3 + i, (4 + i) % 3, True)

    _ddrain()
    for _ in range(2):
        _dwait_ld(0)

    plsc.subcore_barrier()

    pltpu.sync_copy(deg_sh.at[pl.ds(sid * SEG, SEG)],
                    dis_t.at[pl.ds(SEG_OFF, SEG)])

    @pl.loop(0, SEG // 16)
    def _dis(g):
        slc = pl.ds(SEG_OFF + g * 16, 16)
        dis_t[slc] = _fast_rsqrt(dis_t[slc] + jnp.float32(1.0))

    pltpu.sync_copy(dis_t.at[pl.ds(SEG_OFF, SEG)],
                    deg_sh.at[pl.ds(sid * SEG, SEG)])

    @pl.when(jnp.logical_and(cid == 0, sid < last))
    def _():
        pltpu.sync_copy(dis_t.at[pl.ds(SEG_OFF, SEG)],
                        dis_hbm.at[pl.ds(sid * SEG, SEG)])

    @pl.when(jnp.logical_and(cid == 0, sid == last))
    def _():
        pltpu.sync_copy(dis_t.at[pl.ds(SEG_OFF, nrow_last)],
                        dis_hbm.at[pl.ds(last * SEG, nrow_last)])

    plsc.subcore_barrier()

    pltpu.sync_copy(deg_sh.at[pl.ds(0, N_NODES)], dis_t)
    agg_rows = N_EDGES // NC // NS // K
    agg_bch = cid * (N_EDGES // NC // K) + sid * agg_rows
    max_ch2 = N_EDGES // K - 1

    def _ld(c, r):
        b = jnp.minimum(c, max_ch2)
        pltpu.async_copy(pk_hbm.at[1, b], pk.at[r, pl.ds(0, 1)], isem)
        pltpu.async_copy(pk_hbm.at[0, b], pk.at[r, pl.ds(1, 1)], isem)
        pltpu.async_copy(ew_hbm.at[pl.ds(b * K, K)], ewv.at[r], isem)

    def _wait_ld(r):
        pltpu.make_async_copy(pk_hbm.at[0, 0], pk.at[r, pl.ds(0, 1)], isem).wait()
        pltpu.make_async_copy(pk_hbm.at[0, 0], pk.at[r, pl.ds(1, 1)], isem).wait()
        pltpu.make_async_copy(ew_hbm.at[pl.ds(0, K)], ewv.at[r], isem).wait()

    def _gather(ri, rr):
        pltpu.async_copy(x_hbm.at[pk.at[ri, 1]], rows.at[rr], gsem)

    def _wait_gather(rr):
        pltpu.make_async_copy(x_hbm.at[pk.at[0, 1]], rows.at[rr], gsem).wait()

    def _wait_scatter(rr):
        pltpu.make_async_copy(rows.at[rr], agg_sh.at[pk.at[0, 0]], ssem).wait()

    def _agg_chunk(c, i4, i3, scwait):
        if scwait:
            _wait_scatter((i3 + 1) % R3)
        _ld(agg_bch + c + 2, (i4 + 2) % R4)
        _wait_ld((i4 + 1) % R4)
        _gather((i4 + 1) % R4, (i3 + 1) % R3)
        for g in range(K // 16):
            slc = pl.ds(g * 16, 16)
            n16 = (plsc.load_gather(dis_t, [pk[i4, 1, slc]])
                   * plsc.load_gather(dis_t, [pk[i4, 0, slc]])
                   * ewv[i4, slc])
            normv[slc] = n16
        _wait_gather(i3)

        @plsc.parallel_loop(0, K, 1, unroll=8)
        def _scale(j):
            nj = normv[pl.ds(j, 16)][0]
            for cc in range(CH // 16):
                slc = pl.ds(cc * 16, 16)
                rows[i3, j, slc] = rows[i3, j, slc] * nj

        pltpu.async_copy(rows.at[i3], agg_sh.at[pk.at[i4, 0]], ssem, add=True)

    _ld(agg_bch + 0, 0)
    _ld(agg_bch + 1, 1)
    _wait_ld(0)
    _gather(0, 0)
    for c in range(5):
        _agg_chunk(c, c % R4, c % R3, c >= 2)

    @pl.loop(0, (agg_rows - 5) // 12)
    def _agg(g):
        for i in range(12):
            c = 5 + g * 12 + i
            _agg_chunk(c, (5 + i) % R4, (5 + i) % R3, True)

    for r in (0, 1):
        _wait_scatter((r + 123) % R3)
    _wait_gather(125 % R3)
    _wait_ld(0)

    plsc.subcore_barrier()

    @pl.when(sid < last)
    def _():
        pltpu.sync_copy(agg_sh.at[pl.ds(sid * SEG, SEG)],
                        pout_hbm.at[cid, pl.ds(sid * SEG, SEG)])

    @pl.when(sid == last)
    def _():
        pltpu.sync_copy(agg_sh.at[pl.ds(last * SEG, nrow_last)],
                        pout_hbm.at[cid, pl.ds(last * SEG, nrow_last)])


def _sc_aggregate(pk, ew, x):
    mesh = plsc.VectorSubcoreMesh(core_axis_name="c", subcore_axis_name="s")
    f = pl.kernel(
        _sc_body,
        out_type=[
            jax.ShapeDtypeStruct((NC, N_NODES, CH), jnp.float32),
            jax.ShapeDtypeStruct((N_NODES,), jnp.float32),
        ],
        mesh=mesh,
        scratch_types=[
            pltpu.VMEM((R4, 2, K), jnp.int32),
            pltpu.VMEM((3, 10, 1, K), jnp.int32),
            pltpu.VMEM((R4, K), jnp.float32),
            pltpu.VMEM((K + 16,), jnp.float32),
            pltpu.VMEM((R3, K, CH), jnp.float32),
            pltpu.VMEM((N_NODES,), jnp.float32),
            pltpu.VMEM_SHARED((N_PAD,), jnp.float32),
            pltpu.VMEM_SHARED((N_NODES, CH), jnp.float32),
            pltpu.SemaphoreType.DMA,
            pltpu.SemaphoreType.DMA,
            pltpu.SemaphoreType.DMA,
        ],
        compiler_params=pltpu.CompilerParams(needs_layout_passes=False),
    )
    return f(pk, ew, x)


def _tc_body(xb, pa, pb, dis, wt, b, gamma, beta, wft, bf, out):
    agg = pa[0] + pb[0]
    sn = dis[...]
    h = agg + xb[...] * (sn * sn)
    h = lax.dot_general(h, wt[...], (((1,), (1,)), ((), ())),
                        preferred_element_type=jnp.float32) + b[...]
    mu = jnp.mean(h, axis=1, keepdims=True)
    var = jnp.mean((h - mu) * (h - mu), axis=1, keepdims=True)
    h = (h - mu) * lax.rsqrt(var + jnp.float32(EPS)) * gamma[...] + beta[...]
    h = jnp.maximum(h, jnp.float32(0.0))
    out[...] = lax.dot_general(h, wft[...], (((1,), (1,)), ((), ())),
                               preferred_element_type=jnp.float32) + bf[...]


def _tc_tail(x, parts, dis2d, wt, b2, gamma2, beta2, wft, bf2):
    blk = 10000
    grid = N_NODES // blk
    return pl.pallas_call(
        _tc_body,
        grid=(grid,),
        in_specs=[
            pl.BlockSpec((blk, CH), lambda i: (i, 0)),
            pl.BlockSpec((1, blk, CH), lambda i: (0, i, 0)),
            pl.BlockSpec((1, blk, CH), lambda i: (1, i, 0)),
            pl.BlockSpec((blk, 1), lambda i: (i, 0)),
            pl.BlockSpec((CH, CH), lambda i: (0, 0)),
            pl.BlockSpec((1, CH), lambda i: (0, 0)),
            pl.BlockSpec((1, CH), lambda i: (0, 0)),
            pl.BlockSpec((1, CH), lambda i: (0, 0)),
            pl.BlockSpec((16, CH), lambda i: (0, 0)),
            pl.BlockSpec((1, 16), lambda i: (0, 0)),
        ],
        out_specs=pl.BlockSpec((blk, 16), lambda i: (i, 0)),
        out_shape=jax.ShapeDtypeStruct((N_NODES, 16), jnp.float32),
    )(x, parts, parts, dis2d, wt, b2, gamma2, beta2, wft, bf2)


def kernel(x, edge_index, edge_weight, W, b, gamma, beta, Wf, bf):
    pk = edge_index.astype(jnp.int32).reshape(2, N_EDGES // K, 1, K)
    parts, dis = _sc_aggregate(pk, edge_weight, x)
    return _tc_tail(
        x, parts, dis.reshape(N_NODES, 1), W,
        b.reshape(1, CH), gamma.reshape(1, CH), beta.reshape(1, CH),
        Wf, bf.reshape(1, 16),
    )


# speedup vs baseline: 1.0144x; 1.0144x over previous
import functools

import jax
import jax.numpy as jnp
from jax import lax
from jax.experimental import pallas as pl
from jax.experimental.pallas import tpu as pltpu
from jax.experimental.pallas import tpu_sc as plsc

N_NODES = 10000
N_PAD = 10240
SEG = 640
SEG_OFF = 8000
N_EDGES = 320000
CH = 128
K = 80
R3 = 3
R4 = 4
NC = 2
NS = 16
EPS = 1e-5


def _fast_rsqrt(d):
    i = lax.bitcast_convert_type(d, jnp.int32)
    y = lax.bitcast_convert_type(jnp.int32(0x5F3759DF) - (i >> 1), jnp.float32)
    half = jnp.float32(0.5) * d
    for _ in range(3):
        y = y * (jnp.float32(1.5) - half * y * y)
    return y


def _sc_body(pk_hbm, ew_hbm, x_hbm,
             pout_hbm, dis_hbm,
             pk, pkslab, ewv, normv, rows, dis_t,
             deg_sh, agg_sh,
             gsem, ssem, isem):
    cid = lax.axis_index("c")
    sid = lax.axis_index("s")
    zero16 = jnp.zeros((16,), jnp.float32)
    last = NS - 1
    nrow_last = N_NODES - last * SEG
    max_ch = N_EDGES // K - 1

    @pl.loop(0, K * CH // 16)
    def _zrows(i):
        rows[0, i // (CH // 16), pl.ds((i % (CH // 16)) * 16, 16)] = zero16

    @pl.loop(0, SEG // 16)
    def _zseg(i):
        dis_t[pl.ds(SEG_OFF + i * 16, 16)] = zero16

    pltpu.async_copy(dis_t.at[pl.ds(SEG_OFF, SEG)],
                     deg_sh.at[pl.ds(sid * SEG, SEG)], ssem)
    nz = jnp.where(sid < last, SEG // K, nrow_last // K)
    for i in range(SEG // K):
        @pl.when(i < nz)
        def _():
            pltpu.async_copy(rows.at[0],
                             agg_sh.at[pl.ds(sid * SEG + i * K, K)], ssem)
    pltpu.make_async_copy(dis_t.at[pl.ds(SEG_OFF, SEG)],
                          deg_sh.at[pl.ds(0, SEG)], ssem).wait()
    for i in range(SEG // K):
        @pl.when(i < nz)
        def _():
            pltpu.make_async_copy(rows.at[0],
                                  agg_sh.at[pl.ds(0, K)], ssem).wait()

    plsc.subcore_barrier()

    SLB = 10
    deg_slabs = N_EDGES // NS // K // SLB
    deg_sb = sid * deg_slabs
    max_slab = N_EDGES // K // SLB - 1

    def _dld(s, bu):
        ss = jnp.minimum(s, max_slab)
        pltpu.async_copy(pk_hbm.at[1, pl.ds(ss * SLB, SLB)],
                         pkslab.at[bu], isem)
        pltpu.async_copy(ew_hbm.at[pl.ds(ss * SLB * K, SLB * K)],
                         dis_t.at[pl.ds(bu * SLB * K, SLB * K)], isem)

    def _dwait_ld(bu):
        pltpu.make_async_copy(pk_hbm.at[1, pl.ds(0, SLB)],
                              pkslab.at[bu], isem).wait()
        pltpu.make_async_copy(ew_hbm.at[pl.ds(0, SLB * K)],
                              dis_t.at[pl.ds(0, SLB * K)], isem).wait()

    def _dscat(bu):
        for j in range(SLB):
            pltpu.async_copy(dis_t.at[pl.ds((bu * SLB + j) * K, K)],
                             deg_sh.at[pkslab.at[bu, j, 0]], ssem, add=True)

    def _ddrain():
        for j in range(SLB):
            pltpu.make_async_copy(dis_t.at[pl.ds(0, K)],
                                  deg_sh.at[pkslab.at[0, 0, 0]], ssem).wait()

    def _dslab(s, bu, drain):
        _dwait_ld(bu)
        _dscat(bu)
        if drain:
            _ddrain()
        _dld(deg_sb + s + 2, (bu + 2) % 3)

    _dld(deg_sb + 0, 0)
    _dld(deg_sb + 1, 1)
    _dslab(0, 0, False)
    for s in (1, 2, 3):
        _dslab(s, s % 3, True)

    @pl.loop(0, (deg_slabs - 4) // 3)
    def _deg(g):
        for i in range(3):
            _dslab(4 + g * 3 + i, (4 + i) % 3, True)

    _ddrain()
    for _ in range(2):
        _dwait_ld(0)

    plsc.subcore_barrier()

    pltpu.sync_copy(deg_sh.at[pl.ds(sid * SEG, SEG)],
                    dis_t.at[pl.ds(SEG_OFF, SEG)])

    @pl.loop(0, SEG // 16)
    def _dis(g):
        slc = pl.ds(SEG_OFF + g * 16, 16)
        dis_t[slc] = _fast_rsqrt(dis_t[slc] + jnp.float32(1.0))

    pltpu.sync_copy(dis_t.at[pl.ds(SEG_OFF, SEG)],
                    deg_sh.at[pl.ds(sid * SEG, SEG)])

    @pl.when(jnp.logical_and(cid == 0, sid < last))
    def _():
        pltpu.sync_copy(dis_t.at[pl.ds(SEG_OFF, SEG)],
                        dis_hbm.at[pl.ds(sid * SEG, SEG)])

    @pl.when(jnp.logical_and(cid == 0, sid == last))
    def _():
        pltpu.sync_copy(dis_t.at[pl.ds(SEG_OFF, nrow_last)],
                        dis_hbm.at[pl.ds(last * SEG, nrow_last)])

    plsc.subcore_barrier()

    pltpu.sync_copy(deg_sh.at[pl.ds(0, N_NODES)], dis_t)
    agg_rows = N_EDGES // NC // NS // K
    agg_bch = cid * (N_EDGES // NC // K) + sid * agg_rows
    max_ch2 = N_EDGES // K - 1

    def _ld(c, r):
        b = jnp.minimum(c, max_ch2)
        pltpu.async_copy(pk_hbm.at[1, b], pk.at[r, pl.ds(0, 1)], isem)
        pltpu.async_copy(pk_hbm.at[0, b], pk.at[r, pl.ds(1, 1)], isem)
        pltpu.async_copy(ew_hbm.at[pl.ds(b * K, K)], ewv.at[r], isem)

    def _wait_ld(r):
        pltpu.make_async_copy(pk_hbm.at[0, 0], pk.at[r, pl.ds(0, 1)], isem).wait()
        pltpu.make_async_copy(pk_hbm.at[0, 0], pk.at[r, pl.ds(1, 1)], isem).wait()
        pltpu.make_async_copy(ew_hbm.at[pl.ds(0, K)], ewv.at[r], isem).wait()

    def _gather(ri, rr):
        pltpu.async_copy(x_hbm.at[pk.at[ri, 1]], rows.at[rr], gsem)

    def _wait_gather(rr):
        pltpu.make_async_copy(x_hbm.at[pk.at[0, 1]], rows.at[rr], gsem).wait()

    def _wait_scatter(rr):
        pltpu.make_async_copy(rows.at[rr], agg_sh.at[pk.at[0, 0]], ssem).wait()

    def _agg_chunk(c, i4, i3, scwait):
        if scwait:
            _wait_scatter((i3 + 1) % R3)
        _ld(agg_bch + c + 2, (i4 + 2) % R4)
        _wait_ld((i4 + 1) % R4)
        _gather((i4 + 1) % R4, (i3 + 1) % R3)
        for g in range(K // 16):
            slc = pl.ds(g * 16, 16)
            n16 = (plsc.load_gather(dis_t, [pk[i4, 1, slc]])
                   * plsc.load_gather(dis_t, [pk[i4, 0, slc]])
                   * ewv[i4, slc])
            normv[slc] = n16
        _wait_gather(i3)

        @plsc.parallel_loop(0, K, 1, unroll=4)
        def _scale(j):
            nj = normv[pl.ds(j, 16)][0]
            for cc in range(CH // 16):
                slc = pl.ds(cc * 16, 16)
                rows[i3, j, slc] = rows[i3, j, slc] * nj

        pltpu.async_copy(rows.at[i3], agg_sh.at[pk.at[i4, 0]], ssem, add=True)

    _ld(agg_bch + 0, 0)
    _ld(agg_bch + 1, 1)
    _wait_ld(0)
    _gather(0, 0)
    for c in range(5):
        _agg_chunk(c, c % R4, c % R3, c >= 2)

    @pl.loop(0, (agg_rows - 5) // 12)
    def _agg(g):
        for i in range(12):
            c = 5 + g * 12 + i
            _agg_chunk(c, (5 + i) % R4, (5 + i) % R3, True)

    for r in (0, 1):
        _wait_scatter((r + 123) % R3)
    _wait_gather(125 % R3)
    _wait_ld(0)

    plsc.subcore_barrier()

    @pl.when(sid < last)
    def _():
        pltpu.sync_copy(agg_sh.at[pl.ds(sid * SEG, SEG)],
                        pout_hbm.at[cid, pl.ds(sid * SEG, SEG)])

    @pl.when(sid == last)
    def _():
        pltpu.sync_copy(agg_sh.at[pl.ds(last * SEG, nrow_last)],
                        pout_hbm.at[cid, pl.ds(last * SEG, nrow_last)])


def _sc_aggregate(pk, ew, x):
    mesh = plsc.VectorSubcoreMesh(core_axis_name="c", subcore_axis_name="s")
    f = pl.kernel(
        _sc_body,
        out_type=[
            jax.ShapeDtypeStruct((NC, N_NODES, CH), jnp.float32),
            jax.ShapeDtypeStruct((N_NODES,), jnp.float32),
        ],
        mesh=mesh,
        scratch_types=[
            pltpu.VMEM((R4, 2, K), jnp.int32),
            pltpu.VMEM((3, 10, 1, K), jnp.int32),
            pltpu.VMEM((R4, K), jnp.float32),
            pltpu.VMEM((K + 16,), jnp.float32),
            pltpu.VMEM((R3, K, CH), jnp.float32),
            pltpu.VMEM((N_NODES,), jnp.float32),
            pltpu.VMEM_SHARED((N_PAD,), jnp.float32),
            pltpu.VMEM_SHARED((N_NODES, CH), jnp.float32),
            pltpu.SemaphoreType.DMA,
            pltpu.SemaphoreType.DMA,
            pltpu.SemaphoreType.DMA,
        ],
        compiler_params=pltpu.CompilerParams(needs_layout_passes=False),
    )
    return f(pk, ew, x)


def _tc_body(xb, pa, pb, dis, wt, b, gamma, beta, wft, bf, out):
    agg = pa[0] + pb[0]
    sn = dis[...]
    h = agg + xb[...] * (sn * sn)
    h = lax.dot_general(h, wt[...], (((1,), (1,)), ((), ())),
                        preferred_element_type=jnp.float32) + b[...]
    mu = jnp.mean(h, axis=1, keepdims=True)
    var = jnp.mean((h - mu) * (h - mu), axis=1, keepdims=True)
    h = (h - mu) * lax.rsqrt(var + jnp.float32(EPS)) * gamma[...] + beta[...]
    h = jnp.maximum(h, jnp.float32(0.0))
    out[...] = lax.dot_general(h, wft[...], (((1,), (1,)), ((), ())),
                               preferred_element_type=jnp.float32) + bf[...]


def _tc_tail(x, parts, dis2d, wt, b2, gamma2, beta2, wft, bf2):
    blk = 10000
    grid = N_NODES // blk
    return pl.pallas_call(
        _tc_body,
        grid=(grid,),
        in_specs=[
            pl.BlockSpec((blk, CH), lambda i: (i, 0)),
            pl.BlockSpec((1, blk, CH), lambda i: (0, i, 0)),
            pl.BlockSpec((1, blk, CH), lambda i: (1, i, 0)),
            pl.BlockSpec((blk, 1), lambda i: (i, 0)),
            pl.BlockSpec((CH, CH), lambda i: (0, 0)),
            pl.BlockSpec((1, CH), lambda i: (0, 0)),
            pl.BlockSpec((1, CH), lambda i: (0, 0)),
            pl.BlockSpec((1, CH), lambda i: (0, 0)),
            pl.BlockSpec((16, CH), lambda i: (0, 0)),
            pl.BlockSpec((1, 16), lambda i: (0, 0)),
        ],
        out_specs=pl.BlockSpec((blk, 16), lambda i: (i, 0)),
        out_shape=jax.ShapeDtypeStruct((N_NODES, 16), jnp.float32),
    )(x, parts, parts, dis2d, wt, b2, gamma2, beta2, wft, bf2)


def kernel(x, edge_index, edge_weight, W, b, gamma, beta, Wf, bf):
    pk = edge_index.astype(jnp.int32).reshape(2, N_EDGES // K, 1, K)
    parts, dis = _sc_aggregate(pk, edge_weight, x)
    return _tc_tail(
        x, parts, dis.reshape(N_NODES, 1), W,
        b.reshape(1, CH), gamma.reshape(1, CH), beta.reshape(1, CH),
        Wf, bf.reshape(1, 16),
    )
